# initial kernel scaffold (unmeasured)
import jax
import jax.numpy as jnp
from jax import lax
from jax.experimental import pallas as pl
from jax.experimental.pallas import tpu as pltpu

N_DEV = 4
B = 2
SQ = 512
SKV = 512
HQ = 8
DH = 64
DM = 768
DQK = HQ * DH
SCALE = 0.125
NEG = -1e9


def kernel(x, Wq, K_ext, V_ext, Wo):
    K2 = K_ext.reshape(B, SKV, DQK)
    V2 = V_ext.reshape(B, SKV, DQK)

    def body(x_ref, wq_ref, k_ref, v_ref, wo_ref, out_ref,
             abuf, mlbuf, ctx_buf, a_send, a_recv, ml_send, ml_recv):
        my = lax.axis_index("i")
        left = lax.rem(my + N_DEV - 1, N_DEV)
        right = lax.rem(my + 1, N_DEV)

        barrier = pltpu.get_barrier_semaphore()
        for nbr in (left, right):
            pl.semaphore_signal(barrier, inc=1, device_id=(nbr,),
                                device_id_type=pl.DeviceIdType.MESH)
        pl.semaphore_wait(barrier, 2)

        wq = wq_ref[...]
        qs = [jnp.dot(x_ref[b], wq, preferred_element_type=jnp.float32)
              for b in range(B)]

        offset = my * SKV
        i_idx = lax.broadcasted_iota(jnp.int32, (SQ, SKV), 0)
        j_idx = lax.broadcasted_iota(jnp.int32, (SQ, SKV), 1) + offset
        mask = (jnp.abs(i_idx - j_idx) <= 128) | (j_idx < 32) | (i_idx < 32)

        for b in range(B):
            for h in range(HQ):
                q_bh = qs[b][:, h * DH:(h + 1) * DH]
                k_bh = k_ref[b, :, h * DH:(h + 1) * DH]
                v_bh = v_ref[b, :, h * DH:(h + 1) * DH]
                s_mat = lax.dot_general(
                    q_bh, k_bh, (((1,), (1,)), ((), ())),
                    preferred_element_type=jnp.float32) * SCALE
                s_mat = jnp.where(mask, s_mat, NEG)
                m0 = jnp.max(s_mat, axis=1, keepdims=True)
                p = jnp.exp(s_mat - m0)
                l0 = jnp.sum(p, axis=1, keepdims=True)
                acc0 = jnp.dot(p, v_bh, preferred_element_type=jnp.float32)
                abuf[0, b, :, h * DH:(h + 1) * DH] = acc0
                mlbuf[0, b, :, h:h + 1] = m0
                mlbuf[0, b, :, HQ + h:HQ + h + 1] = l0

        for s in range(N_DEV - 1):
            rdmas = []
            for buf, ssem, rsem in ((abuf, a_send, a_recv),
                                    (mlbuf, ml_send, ml_recv)):
                rdma = pltpu.make_async_remote_copy(
                    src_ref=buf.at[s],
                    dst_ref=buf.at[s + 1],
                    send_sem=ssem.at[s],
                    recv_sem=rsem.at[s],
                    device_id=(right,),
                    device_id_type=pl.DeviceIdType.MESH,
                )
                rdma.start()
                rdmas.append(rdma)
            for rdma in rdmas:
                rdma.wait()

        for b in range(B):
            for h in range(HQ):
                ms = [mlbuf[s, b, :, h:h + 1] for s in range(N_DEV)]
                ls = [mlbuf[s, b, :, HQ + h:HQ + h + 1] for s in range(N_DEV)]
                m_f = jnp.maximum(jnp.maximum(ms[0], ms[1]),
                                  jnp.maximum(ms[2], ms[3]))
                l_f = jnp.zeros((SQ, 1), jnp.float32)
                acc_f = jnp.zeros((SQ, DH), jnp.float32)
                for s in range(N_DEV):
                    alpha = jnp.exp(ms[s] - m_f)
                    l_f = l_f + alpha * ls[s]
                    acc_f = acc_f + alpha * abuf[s, b, :, h * DH:(h + 1) * DH]
                ctx_buf[b, :, h * DH:(h + 1) * DH] = acc_f / l_f

        wo = wo_ref[...]
        for b in range(B):
            out_ref[b] = jnp.dot(ctx_buf[b], wo,
                                 preferred_element_type=jnp.float32)

    return pl.pallas_call(
        body,
        out_shape=jax.ShapeDtypeStruct((B, SQ, DM), jnp.float32),
        in_specs=[pl.BlockSpec(memory_space=pltpu.VMEM)] * 5,
        out_specs=pl.BlockSpec(memory_space=pltpu.VMEM),
        scratch_shapes=[
            pltpu.VMEM((N_DEV, B, SQ, DQK), jnp.float32),
            pltpu.VMEM((N_DEV, B, SQ, 2 * HQ), jnp.float32),
            pltpu.VMEM((B, SQ, DQK), jnp.float32),
            pltpu.SemaphoreType.DMA((N_DEV,)),
            pltpu.SemaphoreType.DMA((N_DEV,)),
            pltpu.SemaphoreType.DMA((N_DEV,)),
            pltpu.SemaphoreType.DMA((N_DEV,)),
        ],
        compiler_params=pltpu.CompilerParams(collective_id=0),
    )(x, Wq, K2, V2, Wo)


# baseline (device time: 130501 ns/iter reference)
import jax
import jax.numpy as jnp
from jax import lax
from jax.experimental import pallas as pl
from jax.experimental.pallas import tpu as pltpu

N_DEV = 4
B = 2
SQ = 512
SKV = 512
HQ = 8
DH = 64
DM = 768
DQK = HQ * DH
SCALE = 0.125
NEG = -1e9


def kernel(x, Wq, K_ext, V_ext, Wo):
    K2 = K_ext.reshape(B, SKV, DQK)
    V2 = V_ext.reshape(B, SKV, DQK)

    def body(x_ref, wq_ref, k_ref, v_ref, wo_ref, out_ref,
             abuf, mlbuf, ctx_buf, a_send, a_recv, ml_send, ml_recv):
        my = lax.axis_index("i")
        left = lax.rem(my + N_DEV - 1, N_DEV)
        right = lax.rem(my + 1, N_DEV)

        barrier = pltpu.get_barrier_semaphore()
        for nbr in (left, right):
            pl.semaphore_signal(barrier, inc=1, device_id=(nbr,),
                                device_id_type=pl.DeviceIdType.MESH)
        pl.semaphore_wait(barrier, 2)

        wq = wq_ref[...]
        qs = [jnp.dot(x_ref[b], wq, preferred_element_type=jnp.float32)
              for b in range(B)]

        offset = my * SKV
        i_idx = lax.broadcasted_iota(jnp.int32, (SQ, SKV), 0)
        j_idx = lax.broadcasted_iota(jnp.int32, (SQ, SKV), 1) + offset
        mask = (jnp.abs(i_idx - j_idx) <= 128) | (j_idx < 32) | (i_idx < 32)

        for b in range(B):
            for h in range(HQ):
                q_bh = qs[b][:, h * DH:(h + 1) * DH]
                k_bh = k_ref[b, :, h * DH:(h + 1) * DH]
                v_bh = v_ref[b, :, h * DH:(h + 1) * DH]
                s_mat = lax.dot_general(
                    q_bh, k_bh, (((1,), (1,)), ((), ())),
                    preferred_element_type=jnp.float32) * SCALE
                s_mat = jnp.where(mask, s_mat, NEG)
                m0 = jnp.max(s_mat, axis=1, keepdims=True)
                p = jnp.exp(s_mat - m0)
                l0 = jnp.sum(p, axis=1, keepdims=True)
                acc0 = jnp.dot(p, v_bh, preferred_element_type=jnp.float32)
                abuf[0, b, :, h * DH:(h + 1) * DH] = acc0
                mlbuf[0, b, :, h:h + 1] = m0
                mlbuf[0, b, :, HQ + h:HQ + h + 1] = l0

        for s in range(N_DEV - 1):
            rdmas = []
            for buf, ssem, rsem in ((abuf, a_send, a_recv),
                                    (mlbuf, ml_send, ml_recv)):
                rdma = pltpu.make_async_remote_copy(
                    src_ref=buf.at[s],
                    dst_ref=buf.at[s + 1],
                    send_sem=ssem.at[s],
                    recv_sem=rsem.at[s],
                    device_id=(right,),
                    device_id_type=pl.DeviceIdType.MESH,
                )
                rdma.start()
                rdmas.append(rdma)
            for rdma in rdmas:
                rdma.wait()

        for b in range(B):
            for h in range(HQ):
                ms = [mlbuf[s, b, :, h:h + 1] for s in range(N_DEV)]
                ls = [mlbuf[s, b, :, HQ + h:HQ + h + 1] for s in range(N_DEV)]
                m_f = jnp.maximum(jnp.maximum(ms[0], ms[1]),
                                  jnp.maximum(ms[2], ms[3]))
                l_f = jnp.zeros((SQ, 1), jnp.float32)
                acc_f = jnp.zeros((SQ, DH), jnp.float32)
                for s in range(N_DEV):
                    alpha = jnp.exp(ms[s] - m_f)
                    l_f = l_f + alpha * ls[s]
                    acc_f = acc_f + alpha * abuf[s, b, :, h * DH:(h + 1) * DH]
                ctx_buf[b, :, h * DH:(h + 1) * DH] = acc_f / l_f

        wo = wo_ref[...]
        for b in range(B):
            out_ref[b] = jnp.dot(ctx_buf[b], wo,
                                 preferred_element_type=jnp.float32)

    return pl.pallas_call(
        body,
        out_shape=jax.ShapeDtypeStruct((B, SQ, DM), jnp.float32),
        in_specs=[pl.BlockSpec(memory_space=pltpu.VMEM)] * 5,
        out_specs=pl.BlockSpec(memory_space=pltpu.VMEM),
        scratch_shapes=[
            pltpu.VMEM((N_DEV, B, SQ, DQK), jnp.float32),
            pltpu.VMEM((N_DEV, B, SQ, 2 * HQ), jnp.float32),
            pltpu.VMEM((B, SQ, DQK), jnp.float32),
            pltpu.SemaphoreType.DMA((N_DEV,)),
            pltpu.SemaphoreType.DMA((N_DEV,)),
            pltpu.SemaphoreType.DMA((N_DEV,)),
            pltpu.SemaphoreType.DMA((N_DEV,)),
        ],
        compiler_params=pltpu.CompilerParams(
            collective_id=0, vmem_limit_bytes=100 * 1024 * 1024),
    )(x, Wq, K2, V2, Wo)


# device time: 76107 ns/iter; 1.7147x vs baseline; 1.7147x over previous
import jax
import jax.numpy as jnp
from jax import lax
from jax.experimental import pallas as pl
from jax.experimental.pallas import tpu as pltpu

N_DEV = 4
B = 2
SQ = 512
SKV = 512
HQ = 8
DH = 64
DM = 768
DQK = HQ * DH
QB = SQ // N_DEV
SCALE = 0.125
NEG = -1e9


def kernel(x, Wq, K_ext, V_ext, Wo):
    K2 = K_ext.reshape(B, SKV, DQK)
    V2 = V_ext.reshape(B, SKV, DQK)

    def body(x_ref, wq_ref, k_ref, v_ref, wo_ref, out_ref,
             sacc, sml, racc, rml, cbuf, gacc,
             sa_send, sa_recv, ml_send, ml_recv, g_send, g_recv):
        my = lax.axis_index("i")
        left = lax.rem(my + N_DEV - 1, N_DEV)
        right = lax.rem(my + 1, N_DEV)

        barrier = pltpu.get_barrier_semaphore()
        for nbr in (left, right):
            pl.semaphore_signal(barrier, inc=1, device_id=(nbr,),
                                device_id_type=pl.DeviceIdType.MESH)
        pl.semaphore_wait(barrier, 2)

        wq = wq_ref[...]
        wo = wo_ref[...]

        def partial_block(bk):
            row0 = bk * QB
            i_idx = lax.broadcasted_iota(jnp.int32, (QB, SKV), 0) + row0
            j_idx = lax.broadcasted_iota(jnp.int32, (QB, SKV), 1) + my * SKV
            mask = (jnp.abs(i_idx - j_idx) <= 128) | (j_idx < 32) | (i_idx < 32)
            m_, l_, a_ = [], [], []
            for b in range(B):
                qb = jnp.dot(x_ref[b, pl.ds(row0, QB), :], wq,
                             preferred_element_type=jnp.float32)
                for h in range(HQ):
                    hs = slice(h * DH, (h + 1) * DH)
                    s_mat = lax.dot_general(
                        qb[:, hs], k_ref[b, :, hs], (((1,), (1,)), ((), ())),
                        preferred_element_type=jnp.float32) * SCALE
                    s_mat = jnp.where(mask, s_mat, NEG)
                    m0 = jnp.max(s_mat, axis=1, keepdims=True)
                    p = jnp.exp(s_mat - m0)
                    m_.append(m0)
                    l_.append(jnp.sum(p, axis=1, keepdims=True))
                    a_.append(jnp.dot(p, v_ref[b, :, hs],
                                      preferred_element_type=jnp.float32))
            return m_, l_, a_

        def merge_recv(t, P):
            m_, l_, a_ = P
            om, ol, oa = [], [], []
            for b in range(B):
                for h in range(HQ):
                    i = b * HQ + h
                    hs = slice(h * DH, (h + 1) * DH)
                    m_r = rml[t, b, :, h:h + 1]
                    l_r = rml[t, b, :, HQ + h:HQ + h + 1]
                    a_r = racc[t, b, :, hs]
                    m_f = jnp.maximum(m_[i], m_r)
                    al_p = jnp.exp(m_[i] - m_f)
                    al_r = jnp.exp(m_r - m_f)
                    om.append(m_f)
                    ol.append(al_p * l_[i] + al_r * l_r)
                    oa.append(al_p * a_[i] + al_r * a_r)
            return om, ol, oa

        def store_partial(t, P):
            m_, l_, a_ = P
            for b in range(B):
                for h in range(HQ):
                    i = b * HQ + h
                    sacc[t, b, :, h * DH:(h + 1) * DH] = a_[i]
                    sml[t, b, :, h:h + 1] = m_[i]
                    sml[t, b, :, HQ + h:HQ + h + 1] = l_[i]

        rs_rdmas = []
        P = partial_block(lax.rem(my, N_DEV))
        for t in range(N_DEV - 1):
            store_partial(t, P)
            hop = []
            for buf_s, buf_r, ssem, rsem in (
                    (sacc, racc, sa_send, sa_recv),
                    (sml, rml, ml_send, ml_recv)):
                rdma = pltpu.make_async_remote_copy(
                    src_ref=buf_s.at[t],
                    dst_ref=buf_r.at[t],
                    send_sem=ssem.at[t],
                    recv_sem=rsem.at[t],
                    device_id=(right,),
                    device_id_type=pl.DeviceIdType.MESH,
                )
                rdma.start()
                hop.append(rdma)
            rs_rdmas += hop
            P = partial_block(lax.rem(my - t - 1 + N_DEV, N_DEV))
            for rdma in hop:
                rdma.wait_recv()
            P = merge_recv(t, P)

        for b in range(B):
            for h in range(HQ):
                i = b * HQ + h
                cbuf[b, :, h * DH:(h + 1) * DH] = P[2][i] / P[1][i]

        ag_rdmas = [None] * (N_DEV - 1)
        ag_rdmas[0] = pltpu.make_async_remote_copy(
            src_ref=cbuf, dst_ref=gacc.at[0],
            send_sem=g_send.at[0], recv_sem=g_recv.at[0],
            device_id=(right,), device_id_type=pl.DeviceIdType.MESH,
        )
        ag_rdmas[0].start()

        own_bk = lax.rem(my + 1, N_DEV)
        for b in range(B):
            y = jnp.dot(cbuf[b], wo, preferred_element_type=jnp.float32)
            out_ref[b, pl.ds(own_bk * QB, QB), :] = y

        for h in range(N_DEV - 1):
            ag_rdmas[h].wait_recv()
            if h < N_DEV - 2:
                ag_rdmas[h + 1] = pltpu.make_async_remote_copy(
                    src_ref=gacc.at[h], dst_ref=gacc.at[h + 1],
                    send_sem=g_send.at[h + 1], recv_sem=g_recv.at[h + 1],
                    device_id=(right,), device_id_type=pl.DeviceIdType.MESH,
                )
                ag_rdmas[h + 1].start()
            bk = lax.rem(my - h + N_DEV, N_DEV)
            for b in range(B):
                y = jnp.dot(gacc[h, b], wo, preferred_element_type=jnp.float32)
                out_ref[b, pl.ds(bk * QB, QB), :] = y

        for rdma in rs_rdmas + ag_rdmas:
            rdma.wait_send()

    return pl.pallas_call(
        body,
        out_shape=jax.ShapeDtypeStruct((B, SQ, DM), jnp.float32),
        in_specs=[pl.BlockSpec(memory_space=pltpu.VMEM)] * 5,
        out_specs=pl.BlockSpec(memory_space=pltpu.VMEM),
        scratch_shapes=[
            pltpu.VMEM((N_DEV - 1, B, QB, DQK), jnp.float32),
            pltpu.VMEM((N_DEV - 1, B, QB, 2 * HQ), jnp.float32),
            pltpu.VMEM((N_DEV - 1, B, QB, DQK), jnp.float32),
            pltpu.VMEM((N_DEV - 1, B, QB, 2 * HQ), jnp.float32),
            pltpu.VMEM((B, QB, DQK), jnp.float32),
            pltpu.VMEM((N_DEV - 1, B, QB, DQK), jnp.float32),
            pltpu.SemaphoreType.DMA((N_DEV - 1,)),
            pltpu.SemaphoreType.DMA((N_DEV - 1,)),
            pltpu.SemaphoreType.DMA((N_DEV - 1,)),
            pltpu.SemaphoreType.DMA((N_DEV - 1,)),
            pltpu.SemaphoreType.DMA((N_DEV - 1,)),
            pltpu.SemaphoreType.DMA((N_DEV - 1,)),
        ],
        compiler_params=pltpu.CompilerParams(
            collective_id=0, vmem_limit_bytes=100 * 1024 * 1024),
    )(x, Wq, K2, V2, Wo)


# device time: 61278 ns/iter; 2.1297x vs baseline; 1.2420x over previous
import jax
import jax.numpy as jnp
from jax import lax
from jax.experimental import pallas as pl
from jax.experimental.pallas import tpu as pltpu

N_DEV = 4
B = 2
SQ = 512
SKV = 512
HQ = 8
DH = 64
DM = 768
DQK = HQ * DH
QB = SQ // N_DEV
SCALE = 0.125
MESH = pl.DeviceIdType.MESH


def kernel(x, Wq, K_ext, V_ext, Wo):
    K2 = K_ext.reshape(B, SKV, DQK)
    V2 = V_ext.reshape(B, SKV, DQK)

    def body(x_ref, wq_ref, k_ref, v_ref, wo_ref, out_ref,
             sacc, sl, pacc, plb, cbuf, ybuf,
             p_send, p_recv, l_send, l_recv, y_send, y_recv):
        my = lax.axis_index("i")

        pacc[...] = jnp.zeros((N_DEV, B, QB, DQK), jnp.float32)
        plb[...] = jnp.zeros((N_DEV, B, QB, HQ), jnp.float32)

        barrier = pltpu.get_barrier_semaphore()
        for dd in range(N_DEV):
            @pl.when(dd != my)
            def _():
                pl.semaphore_signal(barrier, inc=1, device_id=(dd,),
                                    device_id_type=MESH)
        pl.semaphore_wait(barrier, N_DEV - 1)

        wq = wq_ref[...]
        wo = wo_ref[...]

        def partial_block(bk):
            row0 = bk * QB
            i_idx = lax.broadcasted_iota(jnp.int32, (QB, SKV), 0) + row0
            j_idx = lax.broadcasted_iota(jnp.int32, (QB, SKV), 1) + my * SKV
            mask = (jnp.abs(i_idx - j_idx) <= 128) | (j_idx < 32) | (i_idx < 32)
            l_, a_ = [], []
            for b in range(B):
                qb = jnp.dot(x_ref[b, pl.ds(row0, QB), :], wq,
                             preferred_element_type=jnp.float32) * SCALE
                for h in range(HQ):
                    hs = slice(h * DH, (h + 1) * DH)
                    s_mat = lax.dot_general(
                        qb[:, hs], k_ref[b, :, hs], (((1,), (1,)), ((), ())),
                        preferred_element_type=jnp.float32)
                    p = jnp.where(mask, jnp.exp(s_mat), 0.0)
                    l_.append(jnp.sum(p, axis=1, keepdims=True))
                    a_.append(jnp.dot(p, v_ref[b, :, hs],
                                      preferred_element_type=jnp.float32))
            return l_, a_

        def valid(o):
            return (my == 0) | (o == 0) | ((my == 1) & (o == 3))

        for o in range(N_DEV):
            @pl.when((o != my) & valid(o))
            def _():
                l_, a_ = partial_block(o)
                for b in range(B):
                    for h in range(HQ):
                        i = b * HQ + h
                        sacc[o, b, :, h * DH:(h + 1) * DH] = a_[i]
                        sl[o, b, :, h:h + 1] = l_[i]
                for buf_s, buf_r, ssem, rsem in (
                        (sacc.at[o], pacc.at[my], p_send, p_recv),
                        (sl.at[o], plb.at[my], l_send, l_recv)):
                    pltpu.make_async_remote_copy(
                        src_ref=buf_s, dst_ref=buf_r,
                        send_sem=ssem.at[o], recv_sem=rsem.at[my],
                        device_id=(o,), device_id_type=MESH,
                    ).start()

        g = jnp.where(my == 0, 1.0, 0.0).astype(jnp.float32)
        l_own, a_own = partial_block(my)

        for s in range(N_DEV):
            @pl.when((s != my) & ((s == 0) | (my == 0) |
                                  ((s == 1) & (my == 3))))
            def _():
                for buf_s, buf_r, ssem, rsem in (
                        (sacc.at[s], pacc.at[s], p_send, p_recv),
                        (sl.at[s], plb.at[s], l_send, l_recv)):
                    pltpu.make_async_remote_copy(
                        src_ref=buf_s, dst_ref=buf_r,
                        send_sem=ssem.at[s], recv_sem=rsem.at[s],
                        device_id=(s,), device_id_type=MESH,
                    ).wait_recv()

        for b in range(B):
            for h in range(HQ):
                i = b * HQ + h
                hs = slice(h * DH, (h + 1) * DH)
                acc = g * a_own[i]
                l_t = g * l_own[i]
                for s in range(N_DEV):
                    acc = acc + pacc[s, b, :, hs]
                    l_t = l_t + plb[s, b, :, h:h + 1]
                cbuf[b, :, hs] = acc / l_t

        own_rows = pl.ds(my * QB, QB)
        for b in range(B):
            y = jnp.dot(cbuf[b], wo, preferred_element_type=jnp.float32)
            ybuf[b] = y
            out_ref[b, own_rows, :] = y

        for dd in range(N_DEV):
            @pl.when(dd != my)
            def _():
                pltpu.make_async_remote_copy(
                    src_ref=ybuf,
                    dst_ref=out_ref.at[:, own_rows, :],
                    send_sem=y_send.at[dd], recv_sem=y_recv.at[my],
                    device_id=(dd,), device_id_type=MESH,
                ).start()

        for s in range(N_DEV):
            @pl.when(s != my)
            def _():
                pltpu.make_async_remote_copy(
                    src_ref=ybuf,
                    dst_ref=out_ref.at[:, pl.ds(s * QB, QB), :],
                    send_sem=y_send.at[s], recv_sem=y_recv.at[s],
                    device_id=(s,), device_id_type=MESH,
                ).wait_recv()

        for o in range(N_DEV):
            @pl.when((o != my) & valid(o))
            def _():
                for buf_s, buf_r, ssem, rsem in (
                        (sacc.at[o], pacc.at[my], p_send, p_recv),
                        (sl.at[o], plb.at[my], l_send, l_recv)):
                    pltpu.make_async_remote_copy(
                        src_ref=buf_s, dst_ref=buf_r,
                        send_sem=ssem.at[o], recv_sem=rsem.at[my],
                        device_id=(o,), device_id_type=MESH,
                    ).wait_send()
        for dd in range(N_DEV):
            @pl.when(dd != my)
            def _():
                pltpu.make_async_remote_copy(
                    src_ref=ybuf,
                    dst_ref=out_ref.at[:, own_rows, :],
                    send_sem=y_send.at[dd], recv_sem=y_recv.at[my],
                    device_id=(dd,), device_id_type=MESH,
                ).wait_send()

    return pl.pallas_call(
        body,
        out_shape=jax.ShapeDtypeStruct((B, SQ, DM), jnp.float32),
        in_specs=[pl.BlockSpec(memory_space=pltpu.VMEM)] * 5,
        out_specs=pl.BlockSpec(memory_space=pltpu.VMEM),
        scratch_shapes=[
            pltpu.VMEM((N_DEV, B, QB, DQK), jnp.float32),
            pltpu.VMEM((N_DEV, B, QB, HQ), jnp.float32),
            pltpu.VMEM((N_DEV, B, QB, DQK), jnp.float32),
            pltpu.VMEM((N_DEV, B, QB, HQ), jnp.float32),
            pltpu.VMEM((B, QB, DQK), jnp.float32),
            pltpu.VMEM((B, QB, DM), jnp.float32),
            pltpu.SemaphoreType.DMA((N_DEV,)),
            pltpu.SemaphoreType.DMA((N_DEV,)),
            pltpu.SemaphoreType.DMA((N_DEV,)),
            pltpu.SemaphoreType.DMA((N_DEV,)),
            pltpu.SemaphoreType.DMA((N_DEV,)),
            pltpu.SemaphoreType.DMA((N_DEV,)),
        ],
        compiler_params=pltpu.CompilerParams(
            collective_id=0, vmem_limit_bytes=100 * 1024 * 1024),
    )(x, Wq, K2, V2, Wo)


# device time: 48470 ns/iter; 2.6924x vs baseline; 1.2642x over previous
import jax
import jax.numpy as jnp
from jax import lax
from jax.experimental import pallas as pl
from jax.experimental.pallas import tpu as pltpu

N_DEV = 4
B = 2
SQ = 512
SKV = 512
HQ = 8
DH = 64
DM = 768
DQK = HQ * DH
QB = SQ // N_DEV
GR = 32
SCALE = 0.125
MESH = pl.DeviceIdType.MESH


def kernel(x, Wq, K_ext, V_ext, Wo):
    K2 = K_ext.reshape(B, SKV, DQK).astype(jnp.bfloat16)
    V2 = V_ext.reshape(B, SKV, DQK).astype(jnp.bfloat16)

    def body(x_ref, wq_ref, k_ref, v_ref, wo_ref, out_ref,
             sacc, sl, pacc, plb, cbuf, gbuf, xbf,
             p_send, p_recv, l_send, l_recv, y_send, y_recv):
        my = lax.axis_index("i")

        pacc[...] = jnp.zeros((N_DEV, B, QB, DQK), jnp.float32)
        plb[...] = jnp.zeros((N_DEV, B, QB, HQ), jnp.float32)

        xbf[...] = x_ref[...].astype(jnp.bfloat16)

        barrier = pltpu.get_barrier_semaphore()
        for dd in range(N_DEV):
            @pl.when(dd != my)
            def _():
                pl.semaphore_signal(barrier, inc=1, device_id=(dd,),
                                    device_id_type=MESH)
        pl.semaphore_wait(barrier, N_DEV - 1)

        bf = jnp.bfloat16
        wq = (wq_ref[...] * SCALE).astype(bf)
        wo = wo_ref[...].astype(bf)

        def qproj(row0, nrows):
            return [
                jnp.dot(xbf[b, pl.ds(row0, nrows), :], wq,
                        preferred_element_type=jnp.float32).astype(bf)
                for b in range(B)
            ]

        def mk_mask(row0, nrows, j0, w):
            i_idx = lax.broadcasted_iota(jnp.int32, (nrows, w), 0) + row0
            j_idx = (lax.broadcasted_iota(jnp.int32, (nrows, w), 1)
                     + my * SKV + j0)
            return ((jnp.abs(i_idx - j_idx) <= 128) | (j_idx < GR)
                    | (i_idx < GR))

        def attn_piece(q_bh, b, hs, j0, w, mask):
            s_mat = lax.dot_general(
                q_bh, k_ref[b, j0:j0 + w, hs],
                (((1,), (1,)), ((), ())),
                preferred_element_type=jnp.float32)
            p = jnp.exp(s_mat)
            if mask is not None:
                p = jnp.where(mask, p, 0.0)
            l = jnp.sum(p, axis=1, keepdims=True)
            acc = jnp.dot(p.astype(bf), v_ref[b, j0:j0 + w, hs],
                          preferred_element_type=jnp.float32)
            return l, acc

        def send_pair(o, nrows, src_dev_is_me_dst, rsem_idx):
            return [
                pltpu.make_async_remote_copy(
                    src_ref=sacc.at[o, :, 0:nrows, :],
                    dst_ref=pacc.at[rsem_idx, :, 0:nrows, :],
                    send_sem=p_send.at[o], recv_sem=p_recv.at[rsem_idx],
                    device_id=(src_dev_is_me_dst,), device_id_type=MESH,
                ),
                pltpu.make_async_remote_copy(
                    src_ref=sl.at[o, :, 0:nrows, :],
                    dst_ref=plb.at[rsem_idx, :, 0:nrows, :],
                    send_sem=l_send.at[o], recv_sem=l_recv.at[rsem_idx],
                    device_id=(src_dev_is_me_dst,), device_id_type=MESH,
                ),
            ]

        @pl.when(my != 0)
        def _():
            qb = qproj(0, GR)
            for b in range(B):
                for h in range(HQ):
                    hs = slice(h * DH, (h + 1) * DH)
                    l, acc = attn_piece(qb[b][:, hs], b, hs, 0, SKV, None)
                    sacc[0, b, 0:GR, hs] = acc
                    sl[0, b, 0:GR, h:h + 1] = l
            for r in send_pair(0, GR, 0, my):
                r.start()

        @pl.when(my == 1)
        def _():
            qb = qproj(3 * QB, QB)
            m3 = mk_mask(3 * QB, QB, 0, QB)
            for b in range(B):
                for h in range(HQ):
                    hs = slice(h * DH, (h + 1) * DH)
                    l, acc = attn_piece(qb[b][:, hs], b, hs, 0, QB, m3)
                    sacc[3, b, :, hs] = acc
                    sl[3, b, :, h:h + 1] = l
            for r in send_pair(3, QB, 3, 1):
                r.start()

        PIECES = {1: ((0, 384),), 2: ((0, GR), (QB, 384)),
                  3: ((0, GR), (2 * QB, 256))}

        @pl.when(my == 0)
        def _():
            for o in (1, 2, 3):
                qb = qproj(o * QB, QB)
                masks = [mk_mask(o * QB, QB, j0, w) for (j0, w) in PIECES[o]]
                for b in range(B):
                    for h in range(HQ):
                        hs = slice(h * DH, (h + 1) * DH)
                        l = jnp.zeros((QB, 1), jnp.float32)
                        acc = jnp.zeros((QB, DH), jnp.float32)
                        for (j0, w), m in zip(PIECES[o], masks):
                            lp, ap = attn_piece(qb[b][:, hs], b, hs,
                                                j0, w, m)
                            l = l + lp
                            acc = acc + ap
                        sacc[o, b, :, hs] = acc
                        sl[o, b, :, h:h + 1] = l
                for r in send_pair(o, QB, o, 0):
                    r.start()
            qb = qproj(0, QB)
            m_b = mk_mask(GR, QB - GR, 0, 2 * QB)
            for b in range(B):
                for h in range(HQ):
                    hs = slice(h * DH, (h + 1) * DH)
                    l_a, a_a = attn_piece(qb[b][0:GR, hs], b, hs,
                                          0, SKV, None)
                    l_b, a_b = attn_piece(qb[b][GR:QB, hs], b, hs,
                                          0, 2 * QB, m_b)
                    pacc[0, b, 0:GR, hs] = a_a
                    pacc[0, b, GR:QB, hs] = a_b
                    plb[0, b, 0:GR, h:h + 1] = l_a
                    plb[0, b, GR:QB, h:h + 1] = l_b

        for s in range(1, N_DEV):
            @pl.when(my == 0)
            def _():
                for r in send_pair(0, GR, 0, s):
                    r.wait_recv()

        @pl.when(my != 0)
        def _():
            for r in send_pair(0, QB, 0, 0):
                r.wait_recv()

        @pl.when(my == 3)
        def _():
            for r in send_pair(3, QB, 3, 1):
                r.wait_recv()

        for b in range(B):
            for h in range(HQ):
                hs = slice(h * DH, (h + 1) * DH)
                acc = pacc[0, b, :, hs]
                l_t = plb[0, b, :, h:h + 1]
                for s in range(1, N_DEV):
                    acc = acc + pacc[s, b, :, hs]
                    l_t = l_t + plb[s, b, :, h:h + 1]
                cbuf[b, :, hs] = (acc / l_t).astype(bf)

        for dd in range(N_DEV):
            @pl.when(dd != my)
            def _():
                pltpu.make_async_remote_copy(
                    src_ref=cbuf,
                    dst_ref=gbuf.at[my],
                    send_sem=y_send.at[dd], recv_sem=y_recv.at[my],
                    device_id=(dd,), device_id_type=MESH,
                ).start()

        own_rows = pl.ds(my * QB, QB)
        for b in range(B):
            out_ref[b, own_rows, :] = jnp.dot(
                cbuf[b], wo, preferred_element_type=jnp.float32)

        for s in (1, 2, 3, 0):
            @pl.when(s != my)
            def _():
                pltpu.make_async_remote_copy(
                    src_ref=cbuf,
                    dst_ref=gbuf.at[s],
                    send_sem=y_send.at[s], recv_sem=y_recv.at[s],
                    device_id=(s,), device_id_type=MESH,
                ).wait_recv()
                for b in range(B):
                    out_ref[b, s * QB:(s + 1) * QB, :] = jnp.dot(
                        gbuf[s, b], wo, preferred_element_type=jnp.float32)

        @pl.when(my != 0)
        def _():
            for r in send_pair(0, GR, 0, my):
                r.wait_send()

        @pl.when(my == 1)
        def _():
            for r in send_pair(3, QB, 3, 1):
                r.wait_send()

        @pl.when(my == 0)
        def _():
            for o in (1, 2, 3):
                for r in send_pair(o, QB, o, 0):
                    r.wait_send()

        for dd in range(N_DEV):
            @pl.when(dd != my)
            def _():
                pltpu.make_async_remote_copy(
                    src_ref=cbuf,
                    dst_ref=gbuf.at[my],
                    send_sem=y_send.at[dd], recv_sem=y_recv.at[my],
                    device_id=(dd,), device_id_type=MESH,
                ).wait_send()

    return pl.pallas_call(
        body,
        out_shape=jax.ShapeDtypeStruct((B, SQ, DM), jnp.float32),
        in_specs=[pl.BlockSpec(memory_space=pltpu.VMEM)] * 5,
        out_specs=pl.BlockSpec(memory_space=pltpu.VMEM),
        scratch_shapes=[
            pltpu.VMEM((N_DEV, B, QB, DQK), jnp.float32),
            pltpu.VMEM((N_DEV, B, QB, HQ), jnp.float32),
            pltpu.VMEM((N_DEV, B, QB, DQK), jnp.float32),
            pltpu.VMEM((N_DEV, B, QB, HQ), jnp.float32),
            pltpu.VMEM((B, QB, DQK), jnp.bfloat16),
            pltpu.VMEM((N_DEV, B, QB, DQK), jnp.bfloat16),
            pltpu.VMEM((B, SQ, DM), jnp.bfloat16),
            pltpu.SemaphoreType.DMA((N_DEV,)),
            pltpu.SemaphoreType.DMA((N_DEV,)),
            pltpu.SemaphoreType.DMA((N_DEV,)),
            pltpu.SemaphoreType.DMA((N_DEV,)),
            pltpu.SemaphoreType.DMA((N_DEV,)),
            pltpu.SemaphoreType.DMA((N_DEV,)),
        ],
        compiler_params=pltpu.CompilerParams(
            collective_id=0, vmem_limit_bytes=100 * 1024 * 1024),
    )(x, Wq, K2, V2, Wo)
